# baseline (speedup 1.0000x reference)
import functools

import jax
import jax.numpy as jnp
from jax import lax
from jax.experimental import pallas as pl
from jax.experimental.pallas import tpu as pltpu
from jax.experimental.pallas import tpu_sc as plsc

B = 1024
E = 32
V = 100000
TV = 2048


@functools.cache
def _sc_gather_fn():
    info = plsc.get_sparse_core_info()
    nw = info.num_cores * info.num_subcores
    b_per_w = B // nw
    mesh = plsc.VectorSubcoreMesh(core_axis_name="c", subcore_axis_name="s")

    @functools.partial(
        pl.kernel,
        mesh=mesh,
        out_type=jax.ShapeDtypeStruct((B, E), jnp.float32),
        scratch_types=[
            pltpu.VMEM((b_per_w,), jnp.int32),
            pltpu.VMEM((b_per_w, E), jnp.float32),
            pltpu.SemaphoreType.DMA,
        ],
        compiler_params=pltpu.CompilerParams(use_tc_tiling_on_sc=False),
    )
    def gather(table_hbm, idx_hbm, out_hbm, idx_v, rows_v, sem):
        wid = lax.axis_index("s") * info.num_cores + lax.axis_index("c")
        base = wid * b_per_w
        pltpu.sync_copy(idx_hbm.at[pl.ds(base, b_per_w)], idx_v)
        pltpu.async_copy(table_hbm.at[idx_v], rows_v, sem).wait()
        pltpu.sync_copy(rows_v, out_hbm.at[pl.ds(base, b_per_w)])

    return gather


def _decoder_body(emb_ref, w_ref, b_ref, out_ref):
    out_ref[...] = (
        jax.lax.dot_general(
            emb_ref[...], w_ref[...], (((1,), (1,)), ((), ())),
            preferred_element_type=jnp.float32,
        )
        + b_ref[...]
    )


def _decoder(emb, W, b2):
    return pl.pallas_call(
        _decoder_body,
        grid=(pl.cdiv(V, TV),),
        in_specs=[
            pl.BlockSpec((B, E), lambda j: (0, 0)),
            pl.BlockSpec((TV, E), lambda j: (j, 0)),
            pl.BlockSpec((1, TV), lambda j: (0, j)),
        ],
        out_specs=pl.BlockSpec((B, TV), lambda j: (0, j)),
        out_shape=jax.ShapeDtypeStruct((B, V), jnp.float32),
    )(emb, W, b2)


def kernel(input_ids, emb_table, W, b):
    ids = input_ids.astype(jnp.int32)
    emb = _sc_gather_fn()(emb_table, ids)
    return _decoder(emb, W, b.reshape(1, V))


# speedup vs baseline: 1.0765x; 1.0765x over previous
import functools

import jax
import jax.numpy as jnp
from jax import lax
from jax.experimental import pallas as pl
from jax.experimental.pallas import tpu as pltpu
from jax.experimental.pallas import tpu_sc as plsc

B = 1024
E = 32
V = 100000
MT = 32


@functools.cache
def _sc_gather_fn():
    info = plsc.get_sparse_core_info()
    nw = info.num_cores * info.num_subcores
    b_per_w = B // nw
    mesh = plsc.VectorSubcoreMesh(core_axis_name="c", subcore_axis_name="s")

    @functools.partial(
        pl.kernel,
        mesh=mesh,
        out_type=jax.ShapeDtypeStruct((B, E), jnp.float32),
        scratch_types=[
            pltpu.VMEM((b_per_w,), jnp.int32),
            pltpu.VMEM((b_per_w, E), jnp.float32),
            pltpu.SemaphoreType.DMA,
        ],
        compiler_params=pltpu.CompilerParams(use_tc_tiling_on_sc=False),
    )
    def gather(table_hbm, idx_hbm, out_hbm, idx_v, rows_v, sem):
        wid = lax.axis_index("s") * info.num_cores + lax.axis_index("c")
        base = wid * b_per_w
        pltpu.sync_copy(idx_hbm.at[pl.ds(base, b_per_w)], idx_v)
        pltpu.async_copy(table_hbm.at[idx_v], rows_v, sem).wait()
        pltpu.sync_copy(rows_v, out_hbm.at[pl.ds(base, b_per_w)])

    return gather


def _decoder_body(emb_ref, wt_ref, b_ref, out_ref):
    out_ref[...] = (
        jnp.dot(emb_ref[...], wt_ref[...], preferred_element_type=jnp.float32)
        + b_ref[...]
    )


def _decoder(emb, Wt, b2):
    return pl.pallas_call(
        _decoder_body,
        grid=(B // MT,),
        in_specs=[
            pl.BlockSpec((MT, E), lambda i: (i, 0)),
            pl.BlockSpec((E, V), lambda i: (0, 0)),
            pl.BlockSpec((1, V), lambda i: (0, 0)),
        ],
        out_specs=pl.BlockSpec((MT, V), lambda i: (i, 0)),
        out_shape=jax.ShapeDtypeStruct((B, V), jnp.float32),
    )(emb, Wt, b2)


def kernel(input_ids, emb_table, W, b):
    ids = input_ids.astype(jnp.int32)
    emb = _sc_gather_fn()(emb_table, ids)
    return _decoder(emb, W.T, b.reshape(1, V))


# speedup vs baseline: 2.4485x; 2.2745x over previous
import functools

import jax
import jax.numpy as jnp
from jax import lax
from jax.experimental import pallas as pl
from jax.experimental.pallas import tpu as pltpu
from jax.experimental.pallas import tpu_sc as plsc

B = 1024
E = 32
V = 100000
VT = 2048


@functools.cache
def _sc_gather_fn():
    info = plsc.get_sparse_core_info()
    nw = info.num_cores * info.num_subcores
    b_per_w = B // nw
    mesh = plsc.VectorSubcoreMesh(core_axis_name="c", subcore_axis_name="s")

    @functools.partial(
        pl.kernel,
        mesh=mesh,
        out_type=jax.ShapeDtypeStruct((B, E), jnp.float32),
        scratch_types=[
            pltpu.VMEM((b_per_w,), jnp.int32),
            pltpu.VMEM((b_per_w, E), jnp.float32),
            pltpu.SemaphoreType.DMA,
        ],
        compiler_params=pltpu.CompilerParams(use_tc_tiling_on_sc=False),
    )
    def gather(table_hbm, idx_hbm, out_hbm, idx_v, rows_v, sem):
        wid = lax.axis_index("s") * info.num_cores + lax.axis_index("c")
        base = wid * b_per_w
        pltpu.sync_copy(idx_hbm.at[pl.ds(base, b_per_w)], idx_v)
        pltpu.async_copy(table_hbm.at[idx_v], rows_v, sem).wait()
        pltpu.sync_copy(rows_v, out_hbm.at[pl.ds(base, b_per_w)])

    return gather


def _decoder_body(w_ref, emb_ref, b_ref, out_ref):
    acc = jax.lax.dot_general(
        w_ref[...], emb_ref[...], (((1,), (1,)), ((), ())),
        preferred_element_type=jnp.float32,
    )
    out_ref[...] = acc + jnp.reshape(b_ref[...], (VT, 1))


def _decoder_t(emb, W, b2):
    return pl.pallas_call(
        _decoder_body,
        grid=(pl.cdiv(V, VT),),
        in_specs=[
            pl.BlockSpec((VT, E), lambda i: (i, 0)),
            pl.BlockSpec((B, E), lambda i: (0, 0)),
            pl.BlockSpec((1, VT), lambda i: (0, i)),
        ],
        out_specs=pl.BlockSpec((VT, B), lambda i: (i, 0)),
        out_shape=jax.ShapeDtypeStruct((V, B), jnp.float32),
    )(W, emb, b2)


def kernel(input_ids, emb_table, W, b):
    ids = input_ids.astype(jnp.int32)
    emb = _sc_gather_fn()(emb_table, ids)
    return _decoder_t(emb, W, b.reshape(1, V)).T


# speedup vs baseline: 2.9774x; 1.2160x over previous
import functools

import jax
import jax.numpy as jnp
from jax import lax
from jax.experimental import pallas as pl
from jax.experimental.pallas import tpu as pltpu
from jax.experimental.pallas import tpu_sc as plsc

B = 1024
E = 32
V = 100000
VT = 2048


@functools.cache
def _sc_gather_fn():
    info = plsc.get_sparse_core_info()
    nw = info.num_cores * info.num_subcores
    b_per_w = B // nw
    mesh = plsc.VectorSubcoreMesh(core_axis_name="c", subcore_axis_name="s")

    @functools.partial(
        pl.kernel,
        mesh=mesh,
        out_type=jax.ShapeDtypeStruct((B, E), jnp.float32),
        scratch_types=[
            pltpu.VMEM((b_per_w,), jnp.int32),
            pltpu.VMEM((b_per_w, E), jnp.float32),
            pltpu.SemaphoreType.DMA,
        ],
        compiler_params=pltpu.CompilerParams(use_tc_tiling_on_sc=False),
    )
    def gather(table_hbm, idx_hbm, out_hbm, idx_v, rows_v, sem):
        wid = lax.axis_index("s") * info.num_cores + lax.axis_index("c")
        base = wid * b_per_w
        pltpu.sync_copy(idx_hbm.at[pl.ds(base, b_per_w)], idx_v)
        pltpu.async_copy(table_hbm.at[idx_v], rows_v, sem).wait()
        pltpu.sync_copy(rows_v, out_hbm.at[pl.ds(base, b_per_w)])

    return gather


def _decoder_body(wt_ref, emb_ref, b_ref, out_ref):
    acc = jax.lax.dot_general(
        wt_ref[...], emb_ref[...], (((0,), (1,)), ((), ())),
        preferred_element_type=jnp.float32,
    )
    out_ref[...] = acc + jnp.reshape(b_ref[...], (VT, 1))


def _decoder_t(emb, Wt, b2):
    return pl.pallas_call(
        _decoder_body,
        grid=(pl.cdiv(V, VT),),
        in_specs=[
            pl.BlockSpec((E, VT), lambda i: (0, i)),
            pl.BlockSpec((B, E), lambda i: (0, 0)),
            pl.BlockSpec((1, VT), lambda i: (0, i)),
        ],
        out_specs=pl.BlockSpec((VT, B), lambda i: (i, 0)),
        out_shape=jax.ShapeDtypeStruct((V, B), jnp.float32),
    )(Wt, emb, b2)


def kernel(input_ids, emb_table, W, b):
    ids = input_ids.astype(jnp.int32)
    emb = _sc_gather_fn()(emb_table, ids)
    return _decoder_t(emb, W.T, b.reshape(1, V)).T
